# Initial kernel scaffold; baseline (speedup 1.0000x reference)
import jax
import jax.numpy as jnp
from jax.experimental import pallas as pl


def kernel(x, edge_index, edge_label_index, W1, b1, W2, b2, Wc1, bc1, Wc2, bc2, Wp, bp):
    raise NotImplementedError("write your pallas kernel here")


# speedup vs baseline: 11.7694x; 11.7694x over previous
import dataclasses
import functools

import jax
import jax.numpy as jnp
from jax import lax
from jax.experimental import pallas as pl
from jax.experimental.pallas import tpu as pltpu
from jax.experimental.pallas import tpu_sc as plsc

NC = 2
NS = 16
NW = NC * NS
LANES = 16


def _sc_params():
    cp = pltpu.CompilerParams()
    if "needs_layout_passes" in pltpu.CompilerParams.__dataclass_fields__:
        cp = dataclasses.replace(cp, needs_layout_passes=False)
    return cp


def _sc_degree(dst2, n_pad):
    nw, ew = dst2.shape
    nr = n_pad // LANES
    mesh = plsc.VectorSubcoreMesh(core_axis_name="core", subcore_axis_name="subcore")

    @functools.partial(
        pl.kernel,
        out_type=jax.ShapeDtypeStruct((nw, nr, LANES), jnp.float32),
        mesh=mesh,
        scratch_types=[
            pltpu.VMEM((ew,), jnp.int32),
            pltpu.VMEM((nr, LANES), jnp.float32),
        ],
        compiler_params=_sc_params(),
    )
    def deg_kernel(dst_hbm, out_hbm, dst_v, hist_v):
        w = lax.axis_index("core") * NS + lax.axis_index("subcore")
        zeros = jnp.zeros((LANES,), jnp.float32)

        @pl.loop(0, nr)
        def _(r):
            hist_v[r, :] = zeros

        pltpu.sync_copy(dst_hbm.at[w], dst_v)
        ones = jnp.ones((LANES,), jnp.float32)

        @pl.loop(0, ew, step=LANES)
        def _(i):
            idx = dst_v[pl.ds(i, LANES)]
            row = lax.shift_right_logical(idx, 4)
            col = lax.bitwise_and(idx, 15)
            plsc.addupdate_scatter(hist_v, [row, col], ones)

        pltpu.sync_copy(hist_v, out_hbm.at[w])

    return deg_kernel(dst2)


def _sc_scatter(gs, pack3, n_acc):
    n, d = gs.shape
    nw, c, k = pack3.shape
    rpt = n_acc // NS
    mesh = plsc.VectorSubcoreMesh(core_axis_name="core", subcore_axis_name="subcore")

    @functools.partial(
        pl.kernel,
        out_type=jax.ShapeDtypeStruct((NC, n_acc, d), jnp.float32),
        mesh=mesh,
        scratch_types=[
            pltpu.VMEM((c, k), jnp.int32),
            pltpu.VMEM((k,), jnp.int32),
            pltpu.VMEM((k,), jnp.int32),
            pltpu.VMEM((1, k), jnp.int32),
            pltpu.VMEM((1, k), jnp.int32),
            pltpu.VMEM((k, d), jnp.float32),
            pltpu.VMEM((k, d), jnp.float32),
            pltpu.VMEM((8, d), jnp.float32),
            pltpu.VMEM_SHARED((n_acc, d), jnp.float32),
            pltpu.SemaphoreType.DMA,
            pltpu.SemaphoreType.DMA,
        ],
        compiler_params=_sc_params(),
    )
    def scat_kernel(gs_hbm, pack_hbm, out_hbm,
                    pack_v, src0_v, src1_v, dst0_v, dst1_v,
                    rows0_v, rows1_v, zrow_v, acc_sh, sem0, sem1):
        cid = lax.axis_index("core")
        sid = lax.axis_index("subcore")
        w = cid * NS + sid
        zeros = jnp.zeros((LANES,), jnp.float32)

        @pl.loop(0, 8)
        def _(r):
            @pl.loop(0, d, step=LANES)
            def _(j):
                zrow_v[r, pl.ds(j, LANES)] = zeros

        @pl.loop(0, rpt, step=8)
        def _(r):
            pltpu.sync_copy(zrow_v, acc_sh.at[pl.ds(sid * rpt + r, 8)])

        pltpu.sync_copy(pack_hbm.at[w], pack_v)

        def unpack(ci, scb, dcb):
            @pl.loop(0, k, step=LANES)
            def _(j):
                pk = pack_v[ci, pl.ds(j, LANES)]
                scb[pl.ds(j, LANES)] = lax.shift_right_logical(pk, 14)
                dcb[0, pl.ds(j, LANES)] = lax.bitwise_and(pk, 16383)

        plsc.subcore_barrier()

        unpack(0, src0_v, dst0_v)
        pltpu.async_copy(gs_hbm.at[src0_v], rows0_v, sem0)

        @pl.loop(0, c, step=2)
        def _(ci):
            pltpu.make_async_copy(gs_hbm.at[src0_v], rows0_v, sem0).wait()

            @pl.when(ci + 1 < c)
            def _():
                unpack(ci + 1, src1_v, dst1_v)
                pltpu.async_copy(gs_hbm.at[src1_v], rows1_v, sem1)

            pltpu.sync_copy(rows0_v, acc_sh.at[dst0_v.at[0]], add=True)

            @pl.when(ci + 1 < c)
            def _():
                pltpu.make_async_copy(gs_hbm.at[src1_v], rows1_v, sem1).wait()

                @pl.when(ci + 2 < c)
                def _():
                    unpack(ci + 2, src0_v, dst0_v)
                    pltpu.async_copy(gs_hbm.at[src0_v], rows0_v, sem0)

                pltpu.sync_copy(rows1_v, acc_sh.at[dst1_v.at[0]], add=True)

        plsc.subcore_barrier()
        pltpu.sync_copy(acc_sh.at[pl.ds(sid * rpt, rpt)],
                        out_hbm.at[cid].at[pl.ds(sid * rpt, rpt)])

    return scat_kernel(gs, pack3)


def _sc_link(tab, lsrc3, ldst3, cvec, d):
    n, dp = tab.shape
    nw, c, k = lsrc3.shape
    e_total = nw * c * k
    ew = c * k
    mesh = plsc.VectorSubcoreMesh(core_axis_name="core", subcore_axis_name="subcore")

    @functools.partial(
        pl.kernel,
        out_type=jax.ShapeDtypeStruct((e_total,), jnp.float32),
        mesh=mesh,
        scratch_types=[
            pltpu.VMEM((c, k), jnp.int32),
            pltpu.VMEM((c, k), jnp.int32),
            pltpu.VMEM((k, dp), jnp.float32),
            pltpu.VMEM((k, dp), jnp.float32),
            pltpu.VMEM((k,), jnp.float32),
            pltpu.VMEM((LANES,), jnp.float32),
            pltpu.SemaphoreType.DMA,
            pltpu.SemaphoreType.DMA,
        ],
        compiler_params=_sc_params(),
    )
    def link_kernel(tab_hbm, lsrc_hbm, ldst_hbm, cvec_hbm, out_hbm,
                    src_v, dst_v, a_v, b_v, o_v, c_v, sem_a, sem_b):
        w = lax.axis_index("core") * NS + lax.axis_index("subcore")
        pltpu.sync_copy(lsrc_hbm.at[w], src_v)
        pltpu.sync_copy(ldst_hbm.at[w], dst_v)
        pltpu.sync_copy(cvec_hbm, c_v)
        iota = lax.iota(jnp.int32, LANES)

        @pl.loop(0, c)
        def _(ci):
            cp_a = pltpu.async_copy(tab_hbm.at[src_v.at[ci]], a_v, sem_a)
            cp_b = pltpu.async_copy(tab_hbm.at[dst_v.at[ci]], b_v, sem_b)
            cp_a.wait()
            cp_b.wait()
            cvec = c_v[pl.ds(0, LANES)]

            @pl.loop(0, k, step=LANES)
            def _(g):
                rows = iota + g

                def fbody(f, acc):
                    cols = jnp.full((LANES,), f, jnp.int32)
                    va = plsc.load_gather(a_v, [rows, cols])
                    vb = plsc.load_gather(b_v, [rows, cols + d])
                    return acc + va * vb

                acc = lax.fori_loop(0, d, fbody, cvec)
                o_v[pl.ds(g, LANES)] = acc

            pltpu.sync_copy(o_v, out_hbm.at[pl.ds(w * ew + ci * k, k)])

    return link_kernel(tab, lsrc3, ldst3, cvec)


_HI = lax.Precision.HIGHEST


def _tc1(x, degT, W1, b1, W2, b2, blk):
    n, din = x.shape
    grid = (n // blk,)

    def body(x_ref, deg_ref, w1_ref, b1_ref, w2_ref, b2_ref,
             h2s_ref, dinv_ref):
        deg = jnp.sum(deg_ref[...], axis=1, keepdims=True) + 1.0
        dinv = lax.rsqrt(deg)
        h = jnp.maximum(jnp.dot(x_ref[...], w1_ref[...], precision=_HI)
                        + b1_ref[...], 0.0)
        h = jnp.maximum(jnp.dot(h, w2_ref[...], precision=_HI)
                        + b2_ref[...], 0.0)
        h2s_ref[...] = h * dinv
        dinv_ref[...] = dinv

    return pl.pallas_call(
        body,
        grid=grid,
        in_specs=[
            pl.BlockSpec((blk, din), lambda i: (i, 0)),
            pl.BlockSpec((blk, degT.shape[1]), lambda i: (i, 0)),
            pl.BlockSpec(W1.shape, lambda i: (0, 0)),
            pl.BlockSpec(b1.shape, lambda i: (0, 0)),
            pl.BlockSpec(W2.shape, lambda i: (0, 0)),
            pl.BlockSpec(b2.shape, lambda i: (0, 0)),
        ],
        out_specs=[
            pl.BlockSpec((blk, W2.shape[1]), lambda i: (i, 0)),
            pl.BlockSpec((blk, 1), lambda i: (i, 0)),
        ],
        out_shape=[
            jax.ShapeDtypeStruct((n, W2.shape[1]), jnp.float32),
            jax.ShapeDtypeStruct((n, 1), jnp.float32),
        ],
    )(x, degT, W1, b1, W2, b2)


def _tc2(parts, h2s, dinv, Wc1, bc1, blk):
    n, d = h2s.shape
    d1 = Wc1.shape[1]
    grid = (n // blk,)

    def body(p_ref, h2s_ref, dinv_ref, wc1_ref, bc1_ref, emb1_ref, e1sp_ref):
        s = p_ref[0] + p_ref[1] + h2s_ref[...]
        pre = jnp.dot(s, wc1_ref[...], precision=_HI)
        emb1 = jnp.maximum(dinv_ref[...] * pre + bc1_ref[...], 0.0)
        emb1_ref[...] = emb1
        e1s = emb1 * dinv_ref[...]
        e1sp_ref[...] = jnp.concatenate(
            [e1s, jnp.zeros((blk, d - d1), jnp.float32)], axis=1)

    return pl.pallas_call(
        body,
        grid=grid,
        in_specs=[
            pl.BlockSpec((2, blk, d), lambda i: (0, i, 0)),
            pl.BlockSpec((blk, d), lambda i: (i, 0)),
            pl.BlockSpec((blk, 1), lambda i: (i, 0)),
            pl.BlockSpec(Wc1.shape, lambda i: (0, 0)),
            pl.BlockSpec(bc1.shape, lambda i: (0, 0)),
        ],
        out_specs=[
            pl.BlockSpec((blk, d1), lambda i: (i, 0)),
            pl.BlockSpec((blk, d), lambda i: (i, 0)),
        ],
        out_shape=[
            jax.ShapeDtypeStruct((n, d1), jnp.float32),
            jax.ShapeDtypeStruct((n, d), jnp.float32),
        ],
    )(parts, h2s, dinv, Wc1, bc1)


def _tc3(parts, emb1, dinv, Wc2, bc2, Wp, bp, blk):
    n, d1 = emb1.shape
    d2 = Wc2.shape[1]
    dp = parts.shape[2]
    grid = (n // blk,)

    def body(p_ref, emb1_ref, dinv_ref, wc2_ref, bc2_ref, wp_ref, bp_ref,
             emb2_ref, emb2w_ref, cvec_ref):
        s = (p_ref[0] + p_ref[1])[:, :d1] + emb1_ref[...] * dinv_ref[...]
        g2 = jnp.dot(s, wc2_ref[...], precision=_HI)
        emb2 = jnp.maximum(dinv_ref[...] * g2 + bc2_ref[...], 0.0)
        emb2_ref[...] = emb2
        wvec = jnp.sum(wp_ref[...], axis=1)
        emb2w_ref[...] = jnp.concatenate(
            [emb2 * wvec[None, :], emb2,
             jnp.zeros((blk, dp - 2 * d2), jnp.float32)], axis=1)

        @pl.when(pl.program_id(0) == 0)
        def _():
            cvec_ref[...] = jnp.full((1, LANES), jnp.sum(bp_ref[...]),
                                     jnp.float32)

    return pl.pallas_call(
        body,
        grid=grid,
        in_specs=[
            pl.BlockSpec((2, blk, dp), lambda i: (0, i, 0)),
            pl.BlockSpec((blk, d1), lambda i: (i, 0)),
            pl.BlockSpec((blk, 1), lambda i: (i, 0)),
            pl.BlockSpec(Wc2.shape, lambda i: (0, 0)),
            pl.BlockSpec(bc2.shape, lambda i: (0, 0)),
            pl.BlockSpec(Wp.shape, lambda i: (0, 0)),
            pl.BlockSpec(bp.shape, lambda i: (0, 0)),
        ],
        out_specs=[
            pl.BlockSpec((blk, d2), lambda i: (i, 0)),
            pl.BlockSpec((blk, dp), lambda i: (i, 0)),
            pl.BlockSpec((1, LANES), lambda i: (0, 0)),
        ],
        out_shape=[
            jax.ShapeDtypeStruct((n, d2), jnp.float32),
            jax.ShapeDtypeStruct((n, dp), jnp.float32),
            jax.ShapeDtypeStruct((1, LANES), jnp.float32),
        ],
    )(parts, emb1, dinv, Wc2, bc2, Wp, bp)


def kernel(x, edge_index, edge_label_index, W1, b1, W2, b2,
           Wc1, bc1, Wc2, bc2, Wp, bp):
    n = x.shape[0]
    e = edge_index.shape[1]
    ew = e // NW
    k = 80
    c = ew // k
    n_pad = ((n + 16 * LANES - 1) // (16 * LANES)) * (16 * LANES)
    blk = 1000

    pack3 = (edge_index[0] * 16384 + edge_index[1]).reshape(NW, c, k)
    dst2 = edge_index[1].reshape(NW, ew)
    lsrc3 = edge_label_index[0].reshape(NW, c, k)
    ldst3 = edge_label_index[1].reshape(NW, c, k)

    deg_parts = _sc_degree(dst2, n_pad)
    degT = deg_parts.reshape(NW, n_pad)[:, :n].T

    h2s, dinv = _tc1(x, degT, W1, b1.reshape(1, -1), W2, b2.reshape(1, -1),
                     blk)
    p1 = _sc_scatter(h2s, pack3, n_pad)
    emb1, e1sp = _tc2(p1, h2s, dinv, Wc1, bc1.reshape(1, -1), blk)
    p2 = _sc_scatter(e1sp, pack3, n_pad)
    emb2, ltab, cvec = _tc3(p2, emb1, dinv, Wc2, bc2.reshape(1, -1), Wp,
                            bp.reshape(1, -1), blk)
    logits = _sc_link(ltab, lsrc3, ldst3, cvec.reshape(LANES), Wc2.shape[1])
    return logits, emb1, emb2


# speedup vs baseline: 14.3598x; 1.2201x over previous
import dataclasses
import functools

import jax
import jax.numpy as jnp
from jax import lax
from jax.experimental import pallas as pl
from jax.experimental.pallas import tpu as pltpu
from jax.experimental.pallas import tpu_sc as plsc

NC = 2
NS = 16
NW = NC * NS
LANES = 16


def _sc_params():
    cp = pltpu.CompilerParams()
    if "needs_layout_passes" in pltpu.CompilerParams.__dataclass_fields__:
        cp = dataclasses.replace(cp, needs_layout_passes=False)
    return cp


def _sc_degree(dst2, n_pad):
    nw, ew = dst2.shape
    nr = n_pad // LANES
    mesh = plsc.VectorSubcoreMesh(core_axis_name="core", subcore_axis_name="subcore")

    @functools.partial(
        pl.kernel,
        out_type=jax.ShapeDtypeStruct((nw, nr, LANES), jnp.float32),
        mesh=mesh,
        scratch_types=[
            pltpu.VMEM((ew,), jnp.int32),
            pltpu.VMEM((nr, LANES), jnp.float32),
        ],
        compiler_params=_sc_params(),
    )
    def deg_kernel(dst_hbm, out_hbm, dst_v, hist_v):
        w = lax.axis_index("core") * NS + lax.axis_index("subcore")
        zeros = jnp.zeros((LANES,), jnp.float32)

        @pl.loop(0, nr)
        def _(r):
            hist_v[r, :] = zeros

        pltpu.sync_copy(dst_hbm.at[w], dst_v)
        ones = jnp.ones((LANES,), jnp.float32)

        @pl.loop(0, ew, step=LANES)
        def _(i):
            idx = dst_v[pl.ds(i, LANES)]
            row = lax.shift_right_logical(idx, 4)
            col = lax.bitwise_and(idx, 15)
            plsc.addupdate_scatter(hist_v, [row, col], ones)

        pltpu.sync_copy(hist_v, out_hbm.at[w])

    return deg_kernel(dst2)


def _sc_scatter(gs, pack3, n_acc):
    n, d = gs.shape
    nw, c, k = pack3.shape
    rpt = n_acc // NS
    mesh = plsc.VectorSubcoreMesh(core_axis_name="core", subcore_axis_name="subcore")

    @functools.partial(
        pl.kernel,
        out_type=jax.ShapeDtypeStruct((NC, n_acc, d), jnp.float32),
        mesh=mesh,
        scratch_types=[
            pltpu.VMEM((c, k), jnp.int32),
            pltpu.VMEM((k,), jnp.int32),
            pltpu.VMEM((k,), jnp.int32),
            pltpu.VMEM((1, k), jnp.int32),
            pltpu.VMEM((1, k), jnp.int32),
            pltpu.VMEM((k, d), jnp.float32),
            pltpu.VMEM((k, d), jnp.float32),
            pltpu.VMEM((8, d), jnp.float32),
            pltpu.VMEM_SHARED((n_acc, d), jnp.float32),
            pltpu.SemaphoreType.DMA,
            pltpu.SemaphoreType.DMA,
        ],
        compiler_params=_sc_params(),
    )
    def scat_kernel(gs_hbm, pack_hbm, out_hbm,
                    pack_v, src0_v, src1_v, dst0_v, dst1_v,
                    rows0_v, rows1_v, zrow_v, acc_sh, sem0, sem1):
        cid = lax.axis_index("core")
        sid = lax.axis_index("subcore")
        w = cid * NS + sid
        zeros = jnp.zeros((LANES,), jnp.float32)

        @pl.loop(0, 8)
        def _(r):
            @pl.loop(0, d, step=LANES)
            def _(j):
                zrow_v[r, pl.ds(j, LANES)] = zeros

        @pl.loop(0, rpt, step=8)
        def _(r):
            pltpu.sync_copy(zrow_v, acc_sh.at[pl.ds(sid * rpt + r, 8)])

        pltpu.sync_copy(pack_hbm.at[w], pack_v)

        def unpack(ci, scb, dcb):
            @pl.loop(0, k, step=LANES)
            def _(j):
                pk = pack_v[ci, pl.ds(j, LANES)]
                scb[pl.ds(j, LANES)] = lax.shift_right_logical(pk, 14)
                dcb[0, pl.ds(j, LANES)] = lax.bitwise_and(pk, 16383)

        plsc.subcore_barrier()

        unpack(0, src0_v, dst0_v)
        pltpu.async_copy(gs_hbm.at[src0_v], rows0_v, sem0)

        @pl.loop(0, c, step=2)
        def _(ci):
            pltpu.make_async_copy(gs_hbm.at[src0_v], rows0_v, sem0).wait()

            @pl.when(ci + 1 < c)
            def _():
                unpack(ci + 1, src1_v, dst1_v)
                pltpu.async_copy(gs_hbm.at[src1_v], rows1_v, sem1)

            pltpu.sync_copy(rows0_v, acc_sh.at[dst0_v.at[0]], add=True)

            @pl.when(ci + 1 < c)
            def _():
                pltpu.make_async_copy(gs_hbm.at[src1_v], rows1_v, sem1).wait()

                @pl.when(ci + 2 < c)
                def _():
                    unpack(ci + 2, src0_v, dst0_v)
                    pltpu.async_copy(gs_hbm.at[src0_v], rows0_v, sem0)

                pltpu.sync_copy(rows1_v, acc_sh.at[dst1_v.at[0]], add=True)

        plsc.subcore_barrier()
        pltpu.sync_copy(acc_sh.at[pl.ds(sid * rpt, rpt)],
                        out_hbm.at[cid].at[pl.ds(sid * rpt, rpt)])

    return scat_kernel(gs, pack3)


def _sc_link(tab, lsrc3, ldst3, cvec, d):
    n, dp = tab.shape
    nw, c, k = lsrc3.shape
    e_total = nw * c * k
    ew = c * k
    mesh = plsc.VectorSubcoreMesh(core_axis_name="core", subcore_axis_name="subcore")

    @functools.partial(
        pl.kernel,
        out_type=jax.ShapeDtypeStruct((e_total,), jnp.float32),
        mesh=mesh,
        scratch_types=[
            pltpu.VMEM((c, k), jnp.int32),
            pltpu.VMEM((c, k), jnp.int32),
            pltpu.VMEM((k, dp), jnp.float32),
            pltpu.VMEM((k, dp), jnp.float32),
            pltpu.VMEM((k, dp), jnp.float32),
            pltpu.VMEM((k, dp), jnp.float32),
            pltpu.VMEM((k,), jnp.float32),
            pltpu.VMEM((LANES,), jnp.float32),
            pltpu.SemaphoreType.DMA,
            pltpu.SemaphoreType.DMA,
            pltpu.SemaphoreType.DMA,
            pltpu.SemaphoreType.DMA,
        ],
        compiler_params=_sc_params(),
    )
    def link_kernel(tab_hbm, lsrc_hbm, ldst_hbm, cvec_hbm, out_hbm,
                    src_v, dst_v, a0_v, b0_v, a1_v, b1_v, o_v, c_v,
                    sem_a0, sem_b0, sem_a1, sem_b1):
        w = lax.axis_index("core") * NS + lax.axis_index("subcore")
        pltpu.sync_copy(lsrc_hbm.at[w], src_v)
        pltpu.sync_copy(ldst_hbm.at[w], dst_v)
        pltpu.sync_copy(cvec_hbm, c_v)
        iota = lax.iota(jnp.int32, LANES)

        def issue(ci, a_v, b_v, sem_a, sem_b):
            pltpu.async_copy(tab_hbm.at[src_v.at[ci]], a_v, sem_a)
            pltpu.async_copy(tab_hbm.at[dst_v.at[ci]], b_v, sem_b)

        def drain(a_v, b_v, sem_a, sem_b):
            pltpu.make_async_copy(tab_hbm.at[src_v.at[0]], a_v, sem_a).wait()
            pltpu.make_async_copy(tab_hbm.at[dst_v.at[0]], b_v, sem_b).wait()

        def compute(ci, a_v, b_v):
            cvec = c_v[pl.ds(0, LANES)]

            @pl.loop(0, k, step=LANES)
            def _(g):
                rows = iota + g
                acc = cvec
                for f in range(d):
                    cols = jnp.full((LANES,), f, jnp.int32)
                    va = plsc.load_gather(a_v, [rows, cols])
                    vb = plsc.load_gather(b_v, [rows, cols + d])
                    acc = acc + va * vb
                o_v[pl.ds(g, LANES)] = acc

            pltpu.sync_copy(o_v, out_hbm.at[pl.ds(w * ew + ci * k, k)])

        issue(0, a0_v, b0_v, sem_a0, sem_b0)

        @pl.loop(0, c, step=2)
        def _(ci):
            drain(a0_v, b0_v, sem_a0, sem_b0)

            @pl.when(ci + 1 < c)
            def _():
                issue(ci + 1, a1_v, b1_v, sem_a1, sem_b1)

            compute(ci, a0_v, b0_v)

            @pl.when(ci + 1 < c)
            def _():
                drain(a1_v, b1_v, sem_a1, sem_b1)

                @pl.when(ci + 2 < c)
                def _():
                    issue(ci + 2, a0_v, b0_v, sem_a0, sem_b0)

                compute(ci + 1, a1_v, b1_v)

    return link_kernel(tab, lsrc3, ldst3, cvec)


_HI = lax.Precision.HIGHEST


def _tc1(x, degT, W1, b1, W2, b2, blk):
    n, din = x.shape
    grid = (n // blk,)

    def body(x_ref, deg_ref, w1_ref, b1_ref, w2_ref, b2_ref,
             h2s_ref, dinv_ref):
        deg = jnp.sum(deg_ref[...], axis=1, keepdims=True) + 1.0
        dinv = lax.rsqrt(deg)
        h = jnp.maximum(jnp.dot(x_ref[...], w1_ref[...], precision=_HI)
                        + b1_ref[...], 0.0)
        h = jnp.maximum(jnp.dot(h, w2_ref[...], precision=_HI)
                        + b2_ref[...], 0.0)
        h2s_ref[...] = h * dinv
        dinv_ref[...] = dinv

    return pl.pallas_call(
        body,
        grid=grid,
        in_specs=[
            pl.BlockSpec((blk, din), lambda i: (i, 0)),
            pl.BlockSpec((blk, degT.shape[1]), lambda i: (i, 0)),
            pl.BlockSpec(W1.shape, lambda i: (0, 0)),
            pl.BlockSpec(b1.shape, lambda i: (0, 0)),
            pl.BlockSpec(W2.shape, lambda i: (0, 0)),
            pl.BlockSpec(b2.shape, lambda i: (0, 0)),
        ],
        out_specs=[
            pl.BlockSpec((blk, W2.shape[1]), lambda i: (i, 0)),
            pl.BlockSpec((blk, 1), lambda i: (i, 0)),
        ],
        out_shape=[
            jax.ShapeDtypeStruct((n, W2.shape[1]), jnp.float32),
            jax.ShapeDtypeStruct((n, 1), jnp.float32),
        ],
    )(x, degT, W1, b1, W2, b2)


def _tc2(parts, h2s, dinv, Wc1, bc1, blk):
    n, d = h2s.shape
    d1 = Wc1.shape[1]
    grid = (n // blk,)

    def body(p_ref, h2s_ref, dinv_ref, wc1_ref, bc1_ref, emb1_ref, e1sp_ref):
        s = p_ref[0] + p_ref[1] + h2s_ref[...]
        pre = jnp.dot(s, wc1_ref[...], precision=_HI)
        emb1 = jnp.maximum(dinv_ref[...] * pre + bc1_ref[...], 0.0)
        emb1_ref[...] = emb1
        e1s = emb1 * dinv_ref[...]
        e1sp_ref[...] = jnp.concatenate(
            [e1s, jnp.zeros((blk, d - d1), jnp.float32)], axis=1)

    return pl.pallas_call(
        body,
        grid=grid,
        in_specs=[
            pl.BlockSpec((2, blk, d), lambda i: (0, i, 0)),
            pl.BlockSpec((blk, d), lambda i: (i, 0)),
            pl.BlockSpec((blk, 1), lambda i: (i, 0)),
            pl.BlockSpec(Wc1.shape, lambda i: (0, 0)),
            pl.BlockSpec(bc1.shape, lambda i: (0, 0)),
        ],
        out_specs=[
            pl.BlockSpec((blk, d1), lambda i: (i, 0)),
            pl.BlockSpec((blk, d), lambda i: (i, 0)),
        ],
        out_shape=[
            jax.ShapeDtypeStruct((n, d1), jnp.float32),
            jax.ShapeDtypeStruct((n, d), jnp.float32),
        ],
    )(parts, h2s, dinv, Wc1, bc1)


def _tc3(parts, emb1, dinv, Wc2, bc2, Wp, bp, blk):
    n, d1 = emb1.shape
    d2 = Wc2.shape[1]
    dp = parts.shape[2]
    grid = (n // blk,)

    def body(p_ref, emb1_ref, dinv_ref, wc2_ref, bc2_ref, wp_ref, bp_ref,
             emb2_ref, emb2w_ref, cvec_ref):
        s = (p_ref[0] + p_ref[1])[:, :d1] + emb1_ref[...] * dinv_ref[...]
        g2 = jnp.dot(s, wc2_ref[...], precision=_HI)
        emb2 = jnp.maximum(dinv_ref[...] * g2 + bc2_ref[...], 0.0)
        emb2_ref[...] = emb2
        wvec = jnp.sum(wp_ref[...], axis=1)
        emb2w_ref[...] = jnp.concatenate(
            [emb2 * wvec[None, :], emb2,
             jnp.zeros((blk, dp - 2 * d2), jnp.float32)], axis=1)

        @pl.when(pl.program_id(0) == 0)
        def _():
            cvec_ref[...] = jnp.full((1, LANES), jnp.sum(bp_ref[...]),
                                     jnp.float32)

    return pl.pallas_call(
        body,
        grid=grid,
        in_specs=[
            pl.BlockSpec((2, blk, dp), lambda i: (0, i, 0)),
            pl.BlockSpec((blk, d1), lambda i: (i, 0)),
            pl.BlockSpec((blk, 1), lambda i: (i, 0)),
            pl.BlockSpec(Wc2.shape, lambda i: (0, 0)),
            pl.BlockSpec(bc2.shape, lambda i: (0, 0)),
            pl.BlockSpec(Wp.shape, lambda i: (0, 0)),
            pl.BlockSpec(bp.shape, lambda i: (0, 0)),
        ],
        out_specs=[
            pl.BlockSpec((blk, d2), lambda i: (i, 0)),
            pl.BlockSpec((blk, dp), lambda i: (i, 0)),
            pl.BlockSpec((1, LANES), lambda i: (0, 0)),
        ],
        out_shape=[
            jax.ShapeDtypeStruct((n, d2), jnp.float32),
            jax.ShapeDtypeStruct((n, dp), jnp.float32),
            jax.ShapeDtypeStruct((1, LANES), jnp.float32),
        ],
    )(parts, emb1, dinv, Wc2, bc2, Wp, bp)


def kernel(x, edge_index, edge_label_index, W1, b1, W2, b2,
           Wc1, bc1, Wc2, bc2, Wp, bp):
    n = x.shape[0]
    e = edge_index.shape[1]
    ew = e // NW
    k = 80
    c = ew // k
    n_pad = ((n + 16 * LANES - 1) // (16 * LANES)) * (16 * LANES)
    blk = 1000

    pack3 = (edge_index[0] * 16384 + edge_index[1]).reshape(NW, c, k)
    dst2 = edge_index[1].reshape(NW, ew)
    lsrc3 = edge_label_index[0].reshape(NW, c, k)
    ldst3 = edge_label_index[1].reshape(NW, c, k)

    deg_parts = _sc_degree(dst2, n_pad)
    degT = deg_parts.reshape(NW, n_pad)[:, :n].T

    h2s, dinv = _tc1(x, degT, W1, b1.reshape(1, -1), W2, b2.reshape(1, -1),
                     blk)
    p1 = _sc_scatter(h2s, pack3, n_pad)
    emb1, e1sp = _tc2(p1, h2s, dinv, Wc1, bc1.reshape(1, -1), blk)
    p2 = _sc_scatter(e1sp, pack3, n_pad)
    emb2, ltab, cvec = _tc3(p2, emb1, dinv, Wc2, bc2.reshape(1, -1), Wp,
                            bp.reshape(1, -1), blk)
    logits = _sc_link(ltab, lsrc3, ldst3, cvec.reshape(LANES), Wc2.shape[1])
    return logits, emb1, emb2


# speedup vs baseline: 18.9418x; 1.3191x over previous
import dataclasses
import functools

import jax
import jax.numpy as jnp
from jax import lax
from jax.experimental import pallas as pl
from jax.experimental.pallas import tpu as pltpu
from jax.experimental.pallas import tpu_sc as plsc

NC = 2
NS = 16
NW = NC * NS
LANES = 16


def _sc_params():
    cp = pltpu.CompilerParams()
    if "needs_layout_passes" in pltpu.CompilerParams.__dataclass_fields__:
        cp = dataclasses.replace(cp, needs_layout_passes=False)
    return cp


def _sc_degree(dst2, n_pad):
    nw, ew = dst2.shape
    nr = n_pad // LANES
    mesh = plsc.VectorSubcoreMesh(core_axis_name="core", subcore_axis_name="subcore")

    @functools.partial(
        pl.kernel,
        out_type=jax.ShapeDtypeStruct((nw, nr, LANES), jnp.float32),
        mesh=mesh,
        scratch_types=[
            pltpu.VMEM((ew,), jnp.int32),
            pltpu.VMEM((nr, LANES), jnp.float32),
        ],
        compiler_params=_sc_params(),
    )
    def deg_kernel(dst_hbm, out_hbm, dst_v, hist_v):
        w = lax.axis_index("core") * NS + lax.axis_index("subcore")
        zeros = jnp.zeros((LANES,), jnp.float32)

        @pl.loop(0, nr)
        def _(r):
            hist_v[r, :] = zeros

        pltpu.sync_copy(dst_hbm.at[w], dst_v)
        ones = jnp.ones((LANES,), jnp.float32)

        @pl.loop(0, ew, step=LANES)
        def _(i):
            idx = dst_v[pl.ds(i, LANES)]
            row = lax.shift_right_logical(idx, 4)
            col = lax.bitwise_and(idx, 15)
            plsc.addupdate_scatter(hist_v, [row, col], ones)

        pltpu.sync_copy(hist_v, out_hbm.at[w])

    return deg_kernel(dst2)


def _sc_scatter(gs, pack3, n_acc):
    n, d = gs.shape
    nw, c, k = pack3.shape
    rpt = n_acc // NS
    mesh = plsc.VectorSubcoreMesh(core_axis_name="core", subcore_axis_name="subcore")

    @functools.partial(
        pl.kernel,
        out_type=jax.ShapeDtypeStruct((NC, n_acc, d), jnp.float32),
        mesh=mesh,
        scratch_types=[
            pltpu.VMEM((c, k), jnp.int32),
            pltpu.VMEM((k,), jnp.int32),
            pltpu.VMEM((k,), jnp.int32),
            pltpu.VMEM((1, k), jnp.int32),
            pltpu.VMEM((1, k), jnp.int32),
            pltpu.VMEM((k, d), jnp.float32),
            pltpu.VMEM((k, d), jnp.float32),
            pltpu.VMEM((8, d), jnp.float32),
            pltpu.VMEM_SHARED((n_acc, d), jnp.float32),
            pltpu.SemaphoreType.DMA,
            pltpu.SemaphoreType.DMA,
        ],
        compiler_params=_sc_params(),
    )
    def scat_kernel(gs_hbm, pack_hbm, out_hbm,
                    pack_v, src0_v, src1_v, dst0_v, dst1_v,
                    rows0_v, rows1_v, zrow_v, acc_sh, sem0, sem1):
        cid = lax.axis_index("core")
        sid = lax.axis_index("subcore")
        w = cid * NS + sid
        zeros = jnp.zeros((LANES,), jnp.float32)

        @pl.loop(0, 8)
        def _(r):
            @pl.loop(0, d, step=LANES)
            def _(j):
                zrow_v[r, pl.ds(j, LANES)] = zeros

        @pl.loop(0, rpt, step=8)
        def _(r):
            pltpu.sync_copy(zrow_v, acc_sh.at[pl.ds(sid * rpt + r, 8)])

        pltpu.sync_copy(pack_hbm.at[w], pack_v)

        def unpack(ci, scb, dcb):
            @pl.loop(0, k, step=LANES)
            def _(j):
                pk = pack_v[ci, pl.ds(j, LANES)]
                scb[pl.ds(j, LANES)] = lax.shift_right_logical(pk, 14)
                dcb[0, pl.ds(j, LANES)] = lax.bitwise_and(pk, 16383)

        plsc.subcore_barrier()

        unpack(0, src0_v, dst0_v)
        pltpu.async_copy(gs_hbm.at[src0_v], rows0_v, sem0)

        @pl.loop(0, c, step=2)
        def _(ci):
            pltpu.make_async_copy(gs_hbm.at[src0_v], rows0_v, sem0).wait()

            @pl.when(ci + 1 < c)
            def _():
                unpack(ci + 1, src1_v, dst1_v)
                pltpu.async_copy(gs_hbm.at[src1_v], rows1_v, sem1)

            pltpu.sync_copy(rows0_v, acc_sh.at[dst0_v.at[0]], add=True)

            @pl.when(ci + 1 < c)
            def _():
                pltpu.make_async_copy(gs_hbm.at[src1_v], rows1_v, sem1).wait()

                @pl.when(ci + 2 < c)
                def _():
                    unpack(ci + 2, src0_v, dst0_v)
                    pltpu.async_copy(gs_hbm.at[src0_v], rows0_v, sem0)

                pltpu.sync_copy(rows1_v, acc_sh.at[dst1_v.at[0]], add=True)

        plsc.subcore_barrier()
        pltpu.sync_copy(acc_sh.at[pl.ds(sid * rpt, rpt)],
                        out_hbm.at[cid].at[pl.ds(sid * rpt, rpt)])

    return scat_kernel(gs, pack3)


def _sc_link2(awT, bT, packl):
    nf, fpt, n = awT.shape
    ngrp, eg = packl.shape
    ke = 2048
    nch = eg // ke
    rpc = ke // 128
    acc_rows = eg // 128
    mesh = plsc.VectorSubcoreMesh(core_axis_name="core", subcore_axis_name="subcore")

    @functools.partial(
        pl.kernel,
        out_type=jax.ShapeDtypeStruct((NC, NS, acc_rows, 128), jnp.float32),
        mesh=mesh,
        scratch_types=[
            pltpu.VMEM((fpt, n), jnp.float32),
            pltpu.VMEM((fpt, n), jnp.float32),
            pltpu.VMEM((ke,), jnp.int32),
            pltpu.VMEM((ke,), jnp.int32),
            pltpu.VMEM((rpc, 128), jnp.float32),
            pltpu.VMEM((rpc, 128), jnp.float32),
            pltpu.SemaphoreType.DMA,
            pltpu.SemaphoreType.DMA,
            pltpu.SemaphoreType.DMA,
            pltpu.SemaphoreType.DMA,
        ],
        compiler_params=_sc_params(),
    )
    def link_kernel(awT_hbm, bT_hbm, packl_hbm, out_hbm,
                    tabw_v, tabb_v, pk0_v, pk1_v, p0_v, p1_v,
                    semi0, semi1, semo0, semo1):
        cid = lax.axis_index("core")
        sid = lax.axis_index("subcore")
        g = lax.shift_right_logical(sid, 3)
        m = lax.bitwise_and(sid, 7)
        grp = cid * 2 + g

        pltpu.sync_copy(awT_hbm.at[m], tabw_v)
        pltpu.sync_copy(bT_hbm.at[m], tabb_v)

        def issue(ci, pk, sem):
            pltpu.async_copy(packl_hbm.at[grp].at[pl.ds(ci * ke, ke)], pk, sem)

        def drain_in(pk, sem):
            pltpu.make_async_copy(packl_hbm.at[grp].at[pl.ds(0, ke)], pk,
                                  sem).wait()

        def out_slab(ci):
            return out_hbm.at[cid].at[sid].at[pl.ds(ci * rpc, rpc)]

        def process(ci, pk, pv, semo, first):
            @pl.when(jnp.logical_not(first))
            def _():
                pltpu.make_async_copy(pv, out_slab(0), semo).wait()

            @pl.loop(0, ke, step=LANES)
            def _(j):
                pk16 = pk[pl.ds(j, LANES)]
                src = lax.shift_right_logical(pk16, 14)
                dst = lax.bitwise_and(pk16, 16383)
                frow = jnp.full((LANES,), 0, jnp.int32)
                acc = plsc.load_gather(tabw_v, [frow, src]) * \
                    plsc.load_gather(tabb_v, [frow, dst])
                for f in range(1, fpt):
                    frow = jnp.full((LANES,), f, jnp.int32)
                    va = plsc.load_gather(tabw_v, [frow, src])
                    vb = plsc.load_gather(tabb_v, [frow, dst])
                    acc = acc + va * vb
                row = lax.div(j, 128)
                lane = lax.rem(j, 128)
                pv[row, pl.ds(lane, LANES)] = acc

            pltpu.async_copy(pv, out_slab(ci), semo)

        issue(0, pk0_v, semi0)

        @pl.loop(0, nch, step=2)
        def _(ci):
            drain_in(pk0_v, semi0)

            @pl.when(ci + 1 < nch)
            def _():
                issue(ci + 1, pk1_v, semi1)

            process(ci, pk0_v, p0_v, semo0, ci == 0)

            @pl.when(ci + 1 < nch)
            def _():
                drain_in(pk1_v, semi1)

                @pl.when(ci + 2 < nch)
                def _():
                    issue(ci + 2, pk0_v, semi0)

                process(ci + 1, pk1_v, p1_v, semo1, ci == 0)

        pltpu.make_async_copy(p0_v, out_slab(0), semo0).wait()
        pltpu.make_async_copy(p1_v, out_slab(0), semo1).wait()

    return link_kernel(awT, bT, packl)


def _tc4(lparts, bp):
    nc2, ns2, rows, lw = lparts.shape

    def body(p_ref, bp_ref, o_ref):
        s = jnp.sum(p_ref[0], axis=0) + jnp.sum(bp_ref[...])
        o_ref[...] = s[None]

    return pl.pallas_call(
        body,
        grid=(4,),
        in_specs=[
            pl.BlockSpec((1, 8, rows, lw),
                         lambda cg: (cg // 2, cg % 2, 0, 0)),
            pl.BlockSpec(bp.shape, lambda cg: (0, 0)),
        ],
        out_specs=pl.BlockSpec((1, rows, lw), lambda cg: (cg, 0, 0)),
        out_shape=jax.ShapeDtypeStruct((4, rows, lw), jnp.float32),
    )(lparts, bp)


_HI = lax.Precision.HIGHEST


def _tc1(x, degT, W1, b1, W2, b2, blk):
    n, din = x.shape
    grid = (n // blk,)

    def body(x_ref, deg_ref, w1_ref, b1_ref, w2_ref, b2_ref,
             h2s_ref, dinv_ref):
        deg = jnp.sum(deg_ref[...], axis=1, keepdims=True) + 1.0
        dinv = lax.rsqrt(deg)
        h = jnp.maximum(jnp.dot(x_ref[...], w1_ref[...], precision=_HI)
                        + b1_ref[...], 0.0)
        h = jnp.maximum(jnp.dot(h, w2_ref[...], precision=_HI)
                        + b2_ref[...], 0.0)
        h2s_ref[...] = h * dinv
        dinv_ref[...] = dinv

    return pl.pallas_call(
        body,
        grid=grid,
        in_specs=[
            pl.BlockSpec((blk, din), lambda i: (i, 0)),
            pl.BlockSpec((blk, degT.shape[1]), lambda i: (i, 0)),
            pl.BlockSpec(W1.shape, lambda i: (0, 0)),
            pl.BlockSpec(b1.shape, lambda i: (0, 0)),
            pl.BlockSpec(W2.shape, lambda i: (0, 0)),
            pl.BlockSpec(b2.shape, lambda i: (0, 0)),
        ],
        out_specs=[
            pl.BlockSpec((blk, W2.shape[1]), lambda i: (i, 0)),
            pl.BlockSpec((blk, 1), lambda i: (i, 0)),
        ],
        out_shape=[
            jax.ShapeDtypeStruct((n, W2.shape[1]), jnp.float32),
            jax.ShapeDtypeStruct((n, 1), jnp.float32),
        ],
    )(x, degT, W1, b1, W2, b2)


def _tc2(parts, h2s, dinv, Wc1, bc1, blk):
    n, d = h2s.shape
    d1 = Wc1.shape[1]
    grid = (n // blk,)

    def body(p_ref, h2s_ref, dinv_ref, wc1_ref, bc1_ref, emb1_ref, e1sp_ref):
        s = p_ref[0] + p_ref[1] + h2s_ref[...]
        pre = jnp.dot(s, wc1_ref[...], precision=_HI)
        emb1 = jnp.maximum(dinv_ref[...] * pre + bc1_ref[...], 0.0)
        emb1_ref[...] = emb1
        e1s = emb1 * dinv_ref[...]
        e1sp_ref[...] = jnp.concatenate(
            [e1s, jnp.zeros((blk, d - d1), jnp.float32)], axis=1)

    return pl.pallas_call(
        body,
        grid=grid,
        in_specs=[
            pl.BlockSpec((2, blk, d), lambda i: (0, i, 0)),
            pl.BlockSpec((blk, d), lambda i: (i, 0)),
            pl.BlockSpec((blk, 1), lambda i: (i, 0)),
            pl.BlockSpec(Wc1.shape, lambda i: (0, 0)),
            pl.BlockSpec(bc1.shape, lambda i: (0, 0)),
        ],
        out_specs=[
            pl.BlockSpec((blk, d1), lambda i: (i, 0)),
            pl.BlockSpec((blk, d), lambda i: (i, 0)),
        ],
        out_shape=[
            jax.ShapeDtypeStruct((n, d1), jnp.float32),
            jax.ShapeDtypeStruct((n, d), jnp.float32),
        ],
    )(parts, h2s, dinv, Wc1, bc1)


def _tc3(parts, emb1, dinv, Wc2, bc2, Wp, blk):
    n, d1 = emb1.shape
    d2 = Wc2.shape[1]
    dp = parts.shape[2]
    grid = (n // blk,)

    def body(p_ref, emb1_ref, dinv_ref, wc2_ref, bc2_ref, wp_ref,
             emb2_ref, awt_ref):
        s = (p_ref[0] + p_ref[1])[:, :d1] + emb1_ref[...] * dinv_ref[...]
        g2 = jnp.dot(s, wc2_ref[...], precision=_HI)
        emb2 = jnp.maximum(dinv_ref[...] * g2 + bc2_ref[...], 0.0)
        emb2_ref[...] = emb2
        wvec = jnp.sum(wp_ref[...], axis=1)
        awt_ref[...] = emb2 * wvec[None, :]

    return pl.pallas_call(
        body,
        grid=grid,
        in_specs=[
            pl.BlockSpec((2, blk, dp), lambda i: (0, i, 0)),
            pl.BlockSpec((blk, d1), lambda i: (i, 0)),
            pl.BlockSpec((blk, 1), lambda i: (i, 0)),
            pl.BlockSpec(Wc2.shape, lambda i: (0, 0)),
            pl.BlockSpec(bc2.shape, lambda i: (0, 0)),
            pl.BlockSpec(Wp.shape, lambda i: (0, 0)),
        ],
        out_specs=[
            pl.BlockSpec((blk, d2), lambda i: (i, 0)),
            pl.BlockSpec((blk, d2), lambda i: (i, 0)),
        ],
        out_shape=[
            jax.ShapeDtypeStruct((n, d2), jnp.float32),
            jax.ShapeDtypeStruct((n, d2), jnp.float32),
        ],
    )(parts, emb1, dinv, Wc2, bc2, Wp)


def kernel(x, edge_index, edge_label_index, W1, b1, W2, b2,
           Wc1, bc1, Wc2, bc2, Wp, bp):
    n = x.shape[0]
    e = edge_index.shape[1]
    ew = e // NW
    k = 80
    c = ew // k
    n_pad = ((n + 16 * LANES - 1) // (16 * LANES)) * (16 * LANES)
    blk = 1000

    pack3 = (edge_index[0] * 16384 + edge_index[1]).reshape(NW, c, k)
    dst2 = edge_index[1].reshape(NW, ew)
    ke = 2048
    eg_real = e // 4
    eg = ((eg_real + ke - 1) // ke) * ke
    packl = jnp.pad(
        (edge_label_index[0] * 16384 + edge_label_index[1]).reshape(4, eg_real),
        ((0, 0), (0, eg - eg_real)))

    deg_parts = _sc_degree(dst2, n_pad)
    degT = deg_parts.reshape(NW, n_pad)[:, :n].T

    h2s, dinv = _tc1(x, degT, W1, b1.reshape(1, -1), W2, b2.reshape(1, -1),
                     blk)
    p1 = _sc_scatter(h2s, pack3, n_pad)
    emb1, e1sp = _tc2(p1, h2s, dinv, Wc1, bc1.reshape(1, -1), blk)
    p2 = _sc_scatter(e1sp, pack3, n_pad)
    emb2, emb2w = _tc3(p2, emb1, dinv, Wc2, bc2.reshape(1, -1), Wp, blk)
    d2 = Wc2.shape[1]
    awT = emb2w.T.reshape(d2 // 4, 4, n)
    bT = emb2.T.reshape(d2 // 4, 4, n)
    lparts = _sc_link2(awT, bT, packl)
    lred = _tc4(lparts, bp.reshape(1, -1))
    logits = lred.reshape(4, eg)[:, :eg_real].reshape(e)
    return logits, emb1, emb2


# speedup vs baseline: 18.9699x; 1.0015x over previous
import dataclasses
import functools

import jax
import jax.numpy as jnp
from jax import lax
from jax.experimental import pallas as pl
from jax.experimental.pallas import tpu as pltpu
from jax.experimental.pallas import tpu_sc as plsc

NC = 2
NS = 16
NW = NC * NS
LANES = 16


def _sc_params():
    cp = pltpu.CompilerParams()
    if "needs_layout_passes" in pltpu.CompilerParams.__dataclass_fields__:
        cp = dataclasses.replace(cp, needs_layout_passes=False)
    return cp


def _sc_degree(dst2, n_pad):
    nw, ew = dst2.shape
    nr = n_pad // LANES
    mesh = plsc.VectorSubcoreMesh(core_axis_name="core", subcore_axis_name="subcore")

    @functools.partial(
        pl.kernel,
        out_type=jax.ShapeDtypeStruct((nw, nr, LANES), jnp.float32),
        mesh=mesh,
        scratch_types=[
            pltpu.VMEM((ew,), jnp.int32),
            pltpu.VMEM((nr, LANES), jnp.float32),
        ],
        compiler_params=_sc_params(),
    )
    def deg_kernel(dst_hbm, out_hbm, dst_v, hist_v):
        w = lax.axis_index("core") * NS + lax.axis_index("subcore")
        zeros = jnp.zeros((LANES,), jnp.float32)

        @pl.loop(0, nr)
        def _(r):
            hist_v[r, :] = zeros

        pltpu.sync_copy(dst_hbm.at[w], dst_v)
        ones = jnp.ones((LANES,), jnp.float32)

        @pl.loop(0, ew, step=LANES)
        def _(i):
            idx = dst_v[pl.ds(i, LANES)]
            row = lax.shift_right_logical(idx, 4)
            col = lax.bitwise_and(idx, 15)
            plsc.addupdate_scatter(hist_v, [row, col], ones)

        pltpu.sync_copy(hist_v, out_hbm.at[w])

    return deg_kernel(dst2)


def _sc_scatter(gs, pack3, n_acc):
    n, d = gs.shape
    nw, c, k = pack3.shape
    rpt = n_acc // NS
    mesh = plsc.VectorSubcoreMesh(core_axis_name="core", subcore_axis_name="subcore")

    @functools.partial(
        pl.kernel,
        out_type=jax.ShapeDtypeStruct((NC, n_acc, d), jnp.float32),
        mesh=mesh,
        scratch_types=[
            pltpu.VMEM((c, k), jnp.int32),
            pltpu.VMEM((k,), jnp.int32),
            pltpu.VMEM((k,), jnp.int32),
            pltpu.VMEM((1, k), jnp.int32),
            pltpu.VMEM((1, k), jnp.int32),
            pltpu.VMEM((k, d), jnp.float32),
            pltpu.VMEM((k, d), jnp.float32),
            pltpu.VMEM((8, d), jnp.float32),
            pltpu.VMEM_SHARED((n_acc, d), jnp.float32),
            pltpu.SemaphoreType.DMA,
            pltpu.SemaphoreType.DMA,
            pltpu.SemaphoreType.DMA,
            pltpu.SemaphoreType.DMA,
        ],
        compiler_params=_sc_params(),
    )
    def scat_kernel(gs_hbm, pack_hbm, out_hbm,
                    pack_v, src0_v, src1_v, dst0_v, dst1_v,
                    rows0_v, rows1_v, zrow_v, acc_sh, sem0, sem1,
                    sema0, sema1):
        cid = lax.axis_index("core")
        sid = lax.axis_index("subcore")
        w = cid * NS + sid
        zeros = jnp.zeros((LANES,), jnp.float32)

        @pl.loop(0, 8)
        def _(r):
            @pl.loop(0, d, step=LANES)
            def _(j):
                zrow_v[r, pl.ds(j, LANES)] = zeros

        @pl.loop(0, rpt, step=8)
        def _(r):
            pltpu.sync_copy(zrow_v, acc_sh.at[pl.ds(sid * rpt + r, 8)])

        pltpu.sync_copy(pack_hbm.at[w], pack_v)

        def unpack(ci, scb, dcb):
            @pl.loop(0, k, step=LANES)
            def _(j):
                pk = pack_v[ci, pl.ds(j, LANES)]
                scb[pl.ds(j, LANES)] = lax.shift_right_logical(pk, 14)
                dcb[0, pl.ds(j, LANES)] = lax.bitwise_and(pk, 16383)

        plsc.subcore_barrier()

        def wait_gather(src_v, rows_v, sem):
            pltpu.make_async_copy(gs_hbm.at[src_v], rows_v, sem).wait()

        def wait_add(rows_v, dst_v, sem):
            pltpu.make_async_copy(rows_v, acc_sh.at[dst_v.at[0]], sem).wait()

        unpack(0, src0_v, dst0_v)
        pltpu.async_copy(gs_hbm.at[src0_v], rows0_v, sem0)

        @pl.loop(0, c, step=2)
        def _(ci):
            wait_gather(src0_v, rows0_v, sem0)
            pltpu.async_copy(rows0_v, acc_sh.at[dst0_v.at[0]], sema0, add=True)

            @pl.when(ci + 1 < c)
            def _():
                @pl.when(ci > 0)
                def _():
                    wait_add(rows1_v, dst1_v, sema1)

                unpack(ci + 1, src1_v, dst1_v)
                pltpu.async_copy(gs_hbm.at[src1_v], rows1_v, sem1)
                wait_gather(src1_v, rows1_v, sem1)
                pltpu.async_copy(rows1_v, acc_sh.at[dst1_v.at[0]], sema1,
                                 add=True)

                @pl.when(ci + 2 < c)
                def _():
                    wait_add(rows0_v, dst0_v, sema0)
                    unpack(ci + 2, src0_v, dst0_v)
                    pltpu.async_copy(gs_hbm.at[src0_v], rows0_v, sem0)

        wait_add(rows0_v, dst0_v, sema0)
        wait_add(rows1_v, dst1_v, sema1)
        plsc.subcore_barrier()
        pltpu.sync_copy(acc_sh.at[pl.ds(sid * rpt, rpt)],
                        out_hbm.at[cid].at[pl.ds(sid * rpt, rpt)])

    return scat_kernel(gs, pack3)


def _sc_link2(awT, bT, packl):
    nf, fpt, n = awT.shape
    ngrp, eg = packl.shape
    ke = 2048
    nch = eg // ke
    rpc = ke // 128
    acc_rows = eg // 128
    mesh = plsc.VectorSubcoreMesh(core_axis_name="core", subcore_axis_name="subcore")

    @functools.partial(
        pl.kernel,
        out_type=jax.ShapeDtypeStruct((NC, NS, acc_rows, 128), jnp.float32),
        mesh=mesh,
        scratch_types=[
            pltpu.VMEM((fpt, n), jnp.float32),
            pltpu.VMEM((fpt, n), jnp.float32),
            pltpu.VMEM((ke,), jnp.int32),
            pltpu.VMEM((ke,), jnp.int32),
            pltpu.VMEM((rpc, 128), jnp.float32),
            pltpu.VMEM((rpc, 128), jnp.float32),
            pltpu.SemaphoreType.DMA,
            pltpu.SemaphoreType.DMA,
            pltpu.SemaphoreType.DMA,
            pltpu.SemaphoreType.DMA,
        ],
        compiler_params=_sc_params(),
    )
    def link_kernel(awT_hbm, bT_hbm, packl_hbm, out_hbm,
                    tabw_v, tabb_v, pk0_v, pk1_v, p0_v, p1_v,
                    semi0, semi1, semo0, semo1):
        cid = lax.axis_index("core")
        sid = lax.axis_index("subcore")
        g = lax.shift_right_logical(sid, 3)
        m = lax.bitwise_and(sid, 7)
        grp = cid * 2 + g

        pltpu.sync_copy(awT_hbm.at[m], tabw_v)
        pltpu.sync_copy(bT_hbm.at[m], tabb_v)

        def issue(ci, pk, sem):
            pltpu.async_copy(packl_hbm.at[grp].at[pl.ds(ci * ke, ke)], pk, sem)

        def drain_in(pk, sem):
            pltpu.make_async_copy(packl_hbm.at[grp].at[pl.ds(0, ke)], pk,
                                  sem).wait()

        def out_slab(ci):
            return out_hbm.at[cid].at[sid].at[pl.ds(ci * rpc, rpc)]

        def process(ci, pk, pv, semo, first):
            @pl.when(jnp.logical_not(first))
            def _():
                pltpu.make_async_copy(pv, out_slab(0), semo).wait()

            @pl.loop(0, ke, step=LANES)
            def _(j):
                pk16 = pk[pl.ds(j, LANES)]
                src = lax.shift_right_logical(pk16, 14)
                dst = lax.bitwise_and(pk16, 16383)
                frow = jnp.full((LANES,), 0, jnp.int32)
                acc = plsc.load_gather(tabw_v, [frow, src]) * \
                    plsc.load_gather(tabb_v, [frow, dst])
                for f in range(1, fpt):
                    frow = jnp.full((LANES,), f, jnp.int32)
                    va = plsc.load_gather(tabw_v, [frow, src])
                    vb = plsc.load_gather(tabb_v, [frow, dst])
                    acc = acc + va * vb
                row = lax.div(j, 128)
                lane = lax.rem(j, 128)
                pv[row, pl.ds(lane, LANES)] = acc

            pltpu.async_copy(pv, out_slab(ci), semo)

        issue(0, pk0_v, semi0)

        @pl.loop(0, nch, step=2)
        def _(ci):
            drain_in(pk0_v, semi0)

            @pl.when(ci + 1 < nch)
            def _():
                issue(ci + 1, pk1_v, semi1)

            process(ci, pk0_v, p0_v, semo0, ci == 0)

            @pl.when(ci + 1 < nch)
            def _():
                drain_in(pk1_v, semi1)

                @pl.when(ci + 2 < nch)
                def _():
                    issue(ci + 2, pk0_v, semi0)

                process(ci + 1, pk1_v, p1_v, semo1, ci == 0)

        pltpu.make_async_copy(p0_v, out_slab(0), semo0).wait()
        pltpu.make_async_copy(p1_v, out_slab(0), semo1).wait()

    return link_kernel(awT, bT, packl)


def _tc4(lparts, bp):
    nc2, ns2, rows, lw = lparts.shape

    def body(p_ref, bp_ref, o_ref):
        s = jnp.sum(p_ref[0], axis=0) + jnp.sum(bp_ref[...])
        o_ref[...] = s[None]

    return pl.pallas_call(
        body,
        grid=(4,),
        in_specs=[
            pl.BlockSpec((1, 8, rows, lw),
                         lambda cg: (cg // 2, cg % 2, 0, 0)),
            pl.BlockSpec(bp.shape, lambda cg: (0, 0)),
        ],
        out_specs=pl.BlockSpec((1, rows, lw), lambda cg: (cg, 0, 0)),
        out_shape=jax.ShapeDtypeStruct((4, rows, lw), jnp.float32),
    )(lparts, bp)


_HI = lax.Precision.HIGHEST


def _tc1(x, degT, W1, b1, W2, b2, blk):
    n, din = x.shape
    grid = (n // blk,)

    def body(x_ref, deg_ref, w1_ref, b1_ref, w2_ref, b2_ref,
             h2s_ref, dinv_ref):
        deg = jnp.sum(deg_ref[...], axis=1, keepdims=True) + 1.0
        dinv = lax.rsqrt(deg)
        h = jnp.maximum(jnp.dot(x_ref[...], w1_ref[...], precision=_HI)
                        + b1_ref[...], 0.0)
        h = jnp.maximum(jnp.dot(h, w2_ref[...], precision=_HI)
                        + b2_ref[...], 0.0)
        h2s_ref[...] = h * dinv
        dinv_ref[...] = dinv

    return pl.pallas_call(
        body,
        grid=grid,
        in_specs=[
            pl.BlockSpec((blk, din), lambda i: (i, 0)),
            pl.BlockSpec((blk, degT.shape[1]), lambda i: (i, 0)),
            pl.BlockSpec(W1.shape, lambda i: (0, 0)),
            pl.BlockSpec(b1.shape, lambda i: (0, 0)),
            pl.BlockSpec(W2.shape, lambda i: (0, 0)),
            pl.BlockSpec(b2.shape, lambda i: (0, 0)),
        ],
        out_specs=[
            pl.BlockSpec((blk, W2.shape[1]), lambda i: (i, 0)),
            pl.BlockSpec((blk, 1), lambda i: (i, 0)),
        ],
        out_shape=[
            jax.ShapeDtypeStruct((n, W2.shape[1]), jnp.float32),
            jax.ShapeDtypeStruct((n, 1), jnp.float32),
        ],
    )(x, degT, W1, b1, W2, b2)


def _tc2(parts, h2s, dinv, Wc1, bc1, blk):
    n, d = h2s.shape
    d1 = Wc1.shape[1]
    grid = (n // blk,)

    def body(p_ref, h2s_ref, dinv_ref, wc1_ref, bc1_ref, emb1_ref, e1sp_ref):
        s = p_ref[0] + p_ref[1] + h2s_ref[...]
        pre = jnp.dot(s, wc1_ref[...], precision=_HI)
        emb1 = jnp.maximum(dinv_ref[...] * pre + bc1_ref[...], 0.0)
        emb1_ref[...] = emb1
        e1s = emb1 * dinv_ref[...]
        e1sp_ref[...] = jnp.concatenate(
            [e1s, jnp.zeros((blk, d - d1), jnp.float32)], axis=1)

    return pl.pallas_call(
        body,
        grid=grid,
        in_specs=[
            pl.BlockSpec((2, blk, d), lambda i: (0, i, 0)),
            pl.BlockSpec((blk, d), lambda i: (i, 0)),
            pl.BlockSpec((blk, 1), lambda i: (i, 0)),
            pl.BlockSpec(Wc1.shape, lambda i: (0, 0)),
            pl.BlockSpec(bc1.shape, lambda i: (0, 0)),
        ],
        out_specs=[
            pl.BlockSpec((blk, d1), lambda i: (i, 0)),
            pl.BlockSpec((blk, d), lambda i: (i, 0)),
        ],
        out_shape=[
            jax.ShapeDtypeStruct((n, d1), jnp.float32),
            jax.ShapeDtypeStruct((n, d), jnp.float32),
        ],
    )(parts, h2s, dinv, Wc1, bc1)


def _tc3(parts, emb1, dinv, Wc2, bc2, Wp, blk):
    n, d1 = emb1.shape
    d2 = Wc2.shape[1]
    dp = parts.shape[2]
    grid = (n // blk,)

    def body(p_ref, emb1_ref, dinv_ref, wc2_ref, bc2_ref, wp_ref,
             emb2_ref, awt_ref):
        s = (p_ref[0] + p_ref[1])[:, :d1] + emb1_ref[...] * dinv_ref[...]
        g2 = jnp.dot(s, wc2_ref[...], precision=_HI)
        emb2 = jnp.maximum(dinv_ref[...] * g2 + bc2_ref[...], 0.0)
        emb2_ref[...] = emb2
        wvec = jnp.sum(wp_ref[...], axis=1)
        awt_ref[...] = emb2 * wvec[None, :]

    return pl.pallas_call(
        body,
        grid=grid,
        in_specs=[
            pl.BlockSpec((2, blk, dp), lambda i: (0, i, 0)),
            pl.BlockSpec((blk, d1), lambda i: (i, 0)),
            pl.BlockSpec((blk, 1), lambda i: (i, 0)),
            pl.BlockSpec(Wc2.shape, lambda i: (0, 0)),
            pl.BlockSpec(bc2.shape, lambda i: (0, 0)),
            pl.BlockSpec(Wp.shape, lambda i: (0, 0)),
        ],
        out_specs=[
            pl.BlockSpec((blk, d2), lambda i: (i, 0)),
            pl.BlockSpec((blk, d2), lambda i: (i, 0)),
        ],
        out_shape=[
            jax.ShapeDtypeStruct((n, d2), jnp.float32),
            jax.ShapeDtypeStruct((n, d2), jnp.float32),
        ],
    )(parts, emb1, dinv, Wc2, bc2, Wp)


def kernel(x, edge_index, edge_label_index, W1, b1, W2, b2,
           Wc1, bc1, Wc2, bc2, Wp, bp):
    n = x.shape[0]
    e = edge_index.shape[1]
    ew = e // NW
    k = 80
    c = ew // k
    n_pad = ((n + 16 * LANES - 1) // (16 * LANES)) * (16 * LANES)
    blk = 1000

    pack3 = (edge_index[0] * 16384 + edge_index[1]).reshape(NW, c, k)
    dst2 = edge_index[1].reshape(NW, ew)
    ke = 2048
    eg_real = e // 4
    eg = ((eg_real + ke - 1) // ke) * ke
    packl = jnp.pad(
        (edge_label_index[0] * 16384 + edge_label_index[1]).reshape(4, eg_real),
        ((0, 0), (0, eg - eg_real)))

    deg_parts = _sc_degree(dst2, n_pad)
    degT = deg_parts.reshape(NW, n_pad)[:, :n].T

    h2s, dinv = _tc1(x, degT, W1, b1.reshape(1, -1), W2, b2.reshape(1, -1),
                     blk)
    p1 = _sc_scatter(h2s, pack3, n_pad)
    emb1, e1sp = _tc2(p1, h2s, dinv, Wc1, bc1.reshape(1, -1), blk)
    p2 = _sc_scatter(e1sp, pack3, n_pad)
    emb2, emb2w = _tc3(p2, emb1, dinv, Wc2, bc2.reshape(1, -1), Wp, blk)
    d2 = Wc2.shape[1]
    awT = emb2w.T.reshape(d2 // 4, 4, n)
    bT = emb2.T.reshape(d2 // 4, 4, n)
    lparts = _sc_link2(awT, bT, packl)
    lred = _tc4(lparts, bp.reshape(1, -1))
    logits = lred.reshape(4, eg)[:, :eg_real].reshape(e)
    return logits, emb1, emb2


# speedup vs baseline: 19.7124x; 1.0391x over previous
import dataclasses
import functools

import jax
import jax.numpy as jnp
from jax import lax
from jax.experimental import pallas as pl
from jax.experimental.pallas import tpu as pltpu
from jax.experimental.pallas import tpu_sc as plsc

NC = 2
NS = 16
NW = NC * NS
LANES = 16


def _sc_params():
    cp = pltpu.CompilerParams()
    if "needs_layout_passes" in pltpu.CompilerParams.__dataclass_fields__:
        cp = dataclasses.replace(cp, needs_layout_passes=False)
    return cp


def _sc_degree(dst2, n_pad):
    nw, ew = dst2.shape
    nr = n_pad // LANES
    mesh = plsc.VectorSubcoreMesh(core_axis_name="core", subcore_axis_name="subcore")

    @functools.partial(
        pl.kernel,
        out_type=jax.ShapeDtypeStruct((nw, nr, LANES), jnp.float32),
        mesh=mesh,
        scratch_types=[
            pltpu.VMEM((ew,), jnp.int32),
            pltpu.VMEM((nr, LANES), jnp.float32),
        ],
        compiler_params=_sc_params(),
    )
    def deg_kernel(dst_hbm, out_hbm, dst_v, hist_v):
        w = lax.axis_index("core") * NS + lax.axis_index("subcore")
        zeros = jnp.zeros((LANES,), jnp.float32)

        @pl.loop(0, nr)
        def _(r):
            hist_v[r, :] = zeros

        pltpu.sync_copy(dst_hbm.at[w], dst_v)
        ones = jnp.ones((LANES,), jnp.float32)

        @pl.loop(0, ew, step=LANES)
        def _(i):
            idx = dst_v[pl.ds(i, LANES)]
            row = lax.shift_right_logical(idx, 4)
            col = lax.bitwise_and(idx, 15)
            plsc.addupdate_scatter(hist_v, [row, col], ones)

        pltpu.sync_copy(hist_v, out_hbm.at[w])

    return deg_kernel(dst2)


def _sc_scatter(gs, pack3, n_acc):
    n, d = gs.shape
    nw, c, k = pack3.shape
    rpt = n_acc // NS
    mesh = plsc.VectorSubcoreMesh(core_axis_name="core", subcore_axis_name="subcore")

    @functools.partial(
        pl.kernel,
        out_type=jax.ShapeDtypeStruct((NC, n_acc, d), jnp.float32),
        mesh=mesh,
        scratch_types=[
            pltpu.VMEM((c, k), jnp.int32),
            pltpu.VMEM((k,), jnp.int32),
            pltpu.VMEM((k,), jnp.int32),
            pltpu.VMEM((1, k), jnp.int32),
            pltpu.VMEM((1, k), jnp.int32),
            pltpu.VMEM((k, d), jnp.float32),
            pltpu.VMEM((k, d), jnp.float32),
            pltpu.VMEM((8, d), jnp.float32),
            pltpu.VMEM_SHARED((n_acc, d), jnp.float32),
            pltpu.SemaphoreType.DMA,
            pltpu.SemaphoreType.DMA,
        ],
        compiler_params=_sc_params(),
    )
    def scat_kernel(gs_hbm, pack_hbm, out_hbm,
                    pack_v, src0_v, src1_v, dst0_v, dst1_v,
                    rows0_v, rows1_v, zrow_v, acc_sh, sem0, sem1):
        cid = lax.axis_index("core")
        sid = lax.axis_index("subcore")
        w = cid * NS + sid
        zeros = jnp.zeros((LANES,), jnp.float32)

        @pl.loop(0, 8)
        def _(r):
            @pl.loop(0, d, step=LANES)
            def _(j):
                zrow_v[r, pl.ds(j, LANES)] = zeros

        @pl.loop(0, rpt, step=8)
        def _(r):
            pltpu.sync_copy(zrow_v, acc_sh.at[pl.ds(sid * rpt + r, 8)])

        pltpu.sync_copy(pack_hbm.at[w], pack_v)

        def unpack(ci, scb, dcb):
            @pl.loop(0, k, step=LANES)
            def _(j):
                pk = pack_v[ci, pl.ds(j, LANES)]
                scb[pl.ds(j, LANES)] = lax.shift_right_logical(pk, 14)
                dcb[0, pl.ds(j, LANES)] = lax.bitwise_and(pk, 16383)

        plsc.subcore_barrier()

        unpack(0, src0_v, dst0_v)
        pltpu.async_copy(gs_hbm.at[src0_v], rows0_v, sem0)

        @pl.loop(0, c, step=2)
        def _(ci):
            pltpu.make_async_copy(gs_hbm.at[src0_v], rows0_v, sem0).wait()

            @pl.when(ci + 1 < c)
            def _():
                unpack(ci + 1, src1_v, dst1_v)
                pltpu.async_copy(gs_hbm.at[src1_v], rows1_v, sem1)

            pltpu.sync_copy(rows0_v, acc_sh.at[dst0_v.at[0]], add=True)

            @pl.when(ci + 1 < c)
            def _():
                pltpu.make_async_copy(gs_hbm.at[src1_v], rows1_v, sem1).wait()

                @pl.when(ci + 2 < c)
                def _():
                    unpack(ci + 2, src0_v, dst0_v)
                    pltpu.async_copy(gs_hbm.at[src0_v], rows0_v, sem0)

                pltpu.sync_copy(rows1_v, acc_sh.at[dst1_v.at[0]], add=True)

        plsc.subcore_barrier()
        pltpu.sync_copy(acc_sh.at[pl.ds(sid * rpt, rpt)],
                        out_hbm.at[cid].at[pl.ds(sid * rpt, rpt)])

    return scat_kernel(gs, pack3)


def _sc_link2(bT, wq3, packl):
    nf, fpt, n = bT.shape
    ngrp, eg = packl.shape
    ke = 2048
    nch = eg // ke
    rpc = ke // 128
    acc_rows = eg // 128
    mesh = plsc.VectorSubcoreMesh(core_axis_name="core", subcore_axis_name="subcore")

    @functools.partial(
        pl.kernel,
        out_type=jax.ShapeDtypeStruct((NC, NS, acc_rows, 128), jnp.float32),
        mesh=mesh,
        scratch_types=[
            pltpu.VMEM((fpt, n), jnp.float32),
            pltpu.VMEM((fpt, LANES), jnp.float32),
            pltpu.VMEM((ke,), jnp.int32),
            pltpu.VMEM((ke,), jnp.int32),
            pltpu.VMEM((rpc, 128), jnp.float32),
            pltpu.VMEM((rpc, 128), jnp.float32),
            pltpu.SemaphoreType.DMA,
            pltpu.SemaphoreType.DMA,
            pltpu.SemaphoreType.DMA,
            pltpu.SemaphoreType.DMA,
        ],
        compiler_params=_sc_params(),
    )
    def link_kernel(bT_hbm, wq_hbm, packl_hbm, out_hbm,
                    tab_v, wq_v, pk0_v, pk1_v, p0_v, p1_v,
                    semi0, semi1, semo0, semo1):
        cid = lax.axis_index("core")
        sid = lax.axis_index("subcore")
        g = lax.shift_right_logical(sid, 3)
        m = lax.bitwise_and(sid, 7)
        grp = cid * 2 + g

        pltpu.sync_copy(bT_hbm.at[m], tab_v)
        pltpu.sync_copy(wq_hbm.at[m], wq_v)

        def issue(ci, pk, sem):
            pltpu.async_copy(packl_hbm.at[grp].at[pl.ds(ci * ke, ke)], pk, sem)

        def drain_in(pk, sem):
            pltpu.make_async_copy(packl_hbm.at[grp].at[pl.ds(0, ke)], pk,
                                  sem).wait()

        def out_slab(ci):
            return out_hbm.at[cid].at[sid].at[pl.ds(ci * rpc, rpc)]

        def process(ci, pk, pv, semo, first):
            @pl.when(jnp.logical_not(first))
            def _():
                pltpu.make_async_copy(pv, out_slab(0), semo).wait()

            @pl.loop(0, ke, step=LANES)
            def _(j):
                pk16 = pk[pl.ds(j, LANES)]
                src = lax.shift_right_logical(pk16, 14)
                dst = lax.bitwise_and(pk16, 16383)
                acc = jnp.zeros((LANES,), jnp.float32)
                for f in range(fpt):
                    frow = jnp.full((LANES,), f, jnp.int32)
                    va = plsc.load_gather(tab_v, [frow, src])
                    vb = plsc.load_gather(tab_v, [frow, dst])
                    ri = plsc.bitcast(va * vb, jnp.uint32)
                    lsb = lax.bitwise_and(
                        lax.shift_right_logical(ri, jnp.uint32(16)),
                        jnp.uint32(1))
                    ri = ri + jnp.uint32(32767) + lsb
                    ri = lax.bitwise_and(ri, jnp.uint32(0xFFFF0000))
                    p = plsc.bitcast(ri, jnp.float32)
                    acc = acc + p * wq_v[f, :]
                row = lax.div(j, 128)
                lane = lax.rem(j, 128)
                pv[row, pl.ds(lane, LANES)] = acc

            pltpu.async_copy(pv, out_slab(ci), semo)

        issue(0, pk0_v, semi0)

        @pl.loop(0, nch, step=2)
        def _(ci):
            drain_in(pk0_v, semi0)

            @pl.when(ci + 1 < nch)
            def _():
                issue(ci + 1, pk1_v, semi1)

            process(ci, pk0_v, p0_v, semo0, ci == 0)

            @pl.when(ci + 1 < nch)
            def _():
                drain_in(pk1_v, semi1)

                @pl.when(ci + 2 < nch)
                def _():
                    issue(ci + 2, pk0_v, semi0)

                process(ci + 1, pk1_v, p1_v, semo1, ci == 0)

        pltpu.make_async_copy(p0_v, out_slab(0), semo0).wait()
        pltpu.make_async_copy(p1_v, out_slab(0), semo1).wait()

    return link_kernel(bT, wq3, packl)


def _tc4(lparts, bp):
    nc2, ns2, rows, lw = lparts.shape

    def body(p_ref, bp_ref, o_ref):
        s = jnp.sum(p_ref[0], axis=0) + jnp.sum(bp_ref[...])
        o_ref[...] = s[None]

    return pl.pallas_call(
        body,
        grid=(4,),
        in_specs=[
            pl.BlockSpec((1, 8, rows, lw),
                         lambda cg: (cg // 2, cg % 2, 0, 0)),
            pl.BlockSpec(bp.shape, lambda cg: (0, 0)),
        ],
        out_specs=pl.BlockSpec((1, rows, lw), lambda cg: (cg, 0, 0)),
        out_shape=jax.ShapeDtypeStruct((4, rows, lw), jnp.float32),
    )(lparts, bp)


_HI = lax.Precision.HIGHEST


def _tc1(x, degT, W1, b1, W2, b2, Wc1, blk):
    n, din = x.shape
    dp = W2.shape[1]
    d1 = Wc1.shape[1]
    grid = (n // blk,)

    def body(x_ref, deg_ref, w1_ref, b1_ref, w2_ref, b2_ref, wc1_ref,
             gs1_ref, dinv_ref):
        deg = jnp.sum(deg_ref[...], axis=1, keepdims=True) + 1.0
        dinv = lax.rsqrt(deg)
        h = jnp.maximum(jnp.dot(x_ref[...], w1_ref[...]) + b1_ref[...], 0.0)
        h = jnp.maximum(jnp.dot(h, w2_ref[...]) + b2_ref[...], 0.0)
        g1 = jnp.dot(h, wc1_ref[...])
        gs1_ref[...] = jnp.concatenate(
            [g1 * dinv, jnp.zeros((blk, dp - d1), jnp.float32)], axis=1)
        dinv_ref[...] = dinv

    return pl.pallas_call(
        body,
        grid=grid,
        in_specs=[
            pl.BlockSpec((blk, din), lambda i: (i, 0)),
            pl.BlockSpec((blk, degT.shape[1]), lambda i: (i, 0)),
            pl.BlockSpec(W1.shape, lambda i: (0, 0)),
            pl.BlockSpec(b1.shape, lambda i: (0, 0)),
            pl.BlockSpec(W2.shape, lambda i: (0, 0)),
            pl.BlockSpec(b2.shape, lambda i: (0, 0)),
            pl.BlockSpec(Wc1.shape, lambda i: (0, 0)),
        ],
        out_specs=[
            pl.BlockSpec((blk, dp), lambda i: (i, 0)),
            pl.BlockSpec((blk, 1), lambda i: (i, 0)),
        ],
        out_shape=[
            jax.ShapeDtypeStruct((n, dp), jnp.float32),
            jax.ShapeDtypeStruct((n, 1), jnp.float32),
        ],
    )(x, degT, W1, b1, W2, b2, Wc1)


def _tc2(parts, gs1, dinv, Wc2, bc1, blk):
    n, d = gs1.shape
    d1 = bc1.shape[1]
    d2 = Wc2.shape[1]
    grid = (n // blk,)

    def body(p_ref, gs1_ref, dinv_ref, wc2_ref, bc1_ref, emb1_ref, gs2_ref):
        s = (p_ref[0] + p_ref[1] + gs1_ref[...])[:, :d1]
        emb1 = jnp.maximum(dinv_ref[...] * s + bc1_ref[...], 0.0)
        emb1_ref[...] = emb1
        g2 = jnp.dot(emb1, wc2_ref[...])
        gs2_ref[...] = jnp.concatenate(
            [g2 * dinv_ref[...], jnp.zeros((blk, d - d2), jnp.float32)],
            axis=1)

    return pl.pallas_call(
        body,
        grid=grid,
        in_specs=[
            pl.BlockSpec((2, blk, d), lambda i: (0, i, 0)),
            pl.BlockSpec((blk, d), lambda i: (i, 0)),
            pl.BlockSpec((blk, 1), lambda i: (i, 0)),
            pl.BlockSpec(Wc2.shape, lambda i: (0, 0)),
            pl.BlockSpec(bc1.shape, lambda i: (0, 0)),
        ],
        out_specs=[
            pl.BlockSpec((blk, d1), lambda i: (i, 0)),
            pl.BlockSpec((blk, d), lambda i: (i, 0)),
        ],
        out_shape=[
            jax.ShapeDtypeStruct((n, d1), jnp.float32),
            jax.ShapeDtypeStruct((n, d), jnp.float32),
        ],
    )(parts, gs1, dinv, Wc2, bc1)


def _tc3(parts, gs2, dinv, bc2, Wp, blk):
    n, dp = gs2.shape
    d2 = Wp.shape[0]
    grid = (n // blk,)

    def body(p_ref, gs2_ref, dinv_ref, bc2_ref, wp_ref,
             emb2_ref, wq_ref):
        s = (p_ref[0] + p_ref[1] + gs2_ref[...])[:, :d2]
        emb2 = jnp.maximum(dinv_ref[...] * s + bc2_ref[...], 0.0)
        emb2_ref[...] = emb2

        @pl.when(pl.program_id(0) == 0)
        def _():
            wpq = wp_ref[...].astype(jnp.bfloat16).astype(jnp.float32)
            wq = jnp.sum(wpq, axis=1)
            wq_ref[...] = jnp.broadcast_to(wq[:, None], (d2, LANES)).reshape(
                d2 // 4, 4, LANES)

    return pl.pallas_call(
        body,
        grid=grid,
        in_specs=[
            pl.BlockSpec((2, blk, dp), lambda i: (0, i, 0)),
            pl.BlockSpec((blk, dp), lambda i: (i, 0)),
            pl.BlockSpec((blk, 1), lambda i: (i, 0)),
            pl.BlockSpec(bc2.shape, lambda i: (0, 0)),
            pl.BlockSpec(Wp.shape, lambda i: (0, 0)),
        ],
        out_specs=[
            pl.BlockSpec((blk, d2), lambda i: (i, 0)),
            pl.BlockSpec((d2 // 4, 4, LANES), lambda i: (0, 0, 0)),
        ],
        out_shape=[
            jax.ShapeDtypeStruct((n, d2), jnp.float32),
            jax.ShapeDtypeStruct((d2 // 4, 4, LANES), jnp.float32),
        ],
    )(parts, gs2, dinv, bc2, Wp)


def kernel(x, edge_index, edge_label_index, W1, b1, W2, b2,
           Wc1, bc1, Wc2, bc2, Wp, bp):
    n = x.shape[0]
    e = edge_index.shape[1]
    ew = e // NW
    k = 80
    c = ew // k
    n_pad = ((n + 16 * LANES - 1) // (16 * LANES)) * (16 * LANES)
    blk = 1000

    pack3 = (edge_index[0] * 16384 + edge_index[1]).reshape(NW, c, k)
    dst2 = edge_index[1].reshape(NW, ew)
    ke = 2048
    eg_real = e // 4
    eg = ((eg_real + ke - 1) // ke) * ke
    packl = jnp.pad(
        (edge_label_index[0] * 16384 + edge_label_index[1]).reshape(4, eg_real),
        ((0, 0), (0, eg - eg_real)))

    deg_parts = _sc_degree(dst2, n_pad)
    degT = deg_parts.reshape(NW, n_pad)[:, :n].T

    gs1, dinv = _tc1(x, degT, W1, b1.reshape(1, -1), W2, b2.reshape(1, -1),
                     Wc1, blk)
    p1 = _sc_scatter(gs1, pack3, n_pad)
    emb1, gs2 = _tc2(p1, gs1, dinv, Wc2, bc1.reshape(1, -1), blk)
    p2 = _sc_scatter(gs2, pack3, n_pad)
    emb2, wq3 = _tc3(p2, gs2, dinv, bc2.reshape(1, -1), Wp, blk)
    d2 = Wp.shape[0]
    bT = emb2.T.reshape(d2 // 4, 4, n)
    lparts = _sc_link2(bT, wq3, packl)
    lred = _tc4(lparts, bp.reshape(1, -1))
    logits = lred.reshape(4, eg)[:, :eg_real].reshape(e)
    return logits, emb1, emb2
